# baseline (speedup 1.0000x reference)
import jax
import jax.numpy as jnp
from jax.experimental import pallas as pl


def _copy_block(v_ref, o_ref):
    o_ref[...] = v_ref[...]


def kernel(video, ques, attr, emb):
    del ques, attr, emb
    b, t, d = video.shape
    block_b = 128
    out = pl.pallas_call(
        _copy_block,
        grid=(b // block_b,),
        in_specs=[pl.BlockSpec((block_b, t, d), lambda i: (i, 0, 0))],
        out_specs=pl.BlockSpec((block_b, t, d), lambda i: (i, 0, 0)),
        out_shape=jax.ShapeDtypeStruct(video.shape, video.dtype),
    )(video)
    return out
